# Initial kernel scaffold; baseline (speedup 1.0000x reference)
import jax
import jax.numpy as jnp
from jax.experimental import pallas as pl


def kernel(h, edge_index, W1, b1, W2, b2, W3, b3, W4, b4):
    raise NotImplementedError("write your pallas kernel here")


# speedup vs baseline: 8.6713x; 8.6713x over previous
import functools

import jax
import jax.numpy as jnp
from jax import lax
from jax.experimental import pallas as pl
from jax.experimental.pallas import tpu as pltpu
from jax.experimental.pallas import tpu_sc as plsc

N_NODES = 10000
N_EDGES = 320000
IN_DIM = 128
HIDDEN = 64

NC = 2
NS = 16
NW = NC * NS

BURST = 128
NB = 79
EDGES_PER_TILE = NB * BURST
E_PAD = NW * EDGES_PER_TILE
N_PAD = 10112
ROWS_PER_TILE = N_PAD // NS
DEG_ROWS = 640
N_DEG = DEG_ROWS * 16
DEG_ROWS_PER_TILE = DEG_ROWS // NS
SETTLE_NS = 3000


def _msg_body(h_ref, w1_ref, b1_ref, w2_ref, b2_ref, out_ref):
    m1 = jnp.dot(h_ref[...], w1_ref[...], preferred_element_type=jnp.float32)
    m1 = jnp.maximum(m1 + b1_ref[...], 0.0)
    m2 = jnp.dot(m1, w2_ref[...], preferred_element_type=jnp.float32)
    out_ref[...] = jnp.maximum(m2 + b2_ref[...], 0.0)


def _upd_body(h_ref, pa_ref, dd_ref, w3a_ref, w3b_ref, b3_ref, w4_ref, b4_ref,
              out_ref):
    p = pa_ref[0, :N_NODES, :] + pa_ref[1, :N_NODES, :]
    deg = jnp.maximum(dd_ref[:, 0:1] + dd_ref[:, 1:2], 1.0)
    agg = p / deg
    z = jnp.dot(h_ref[...], w3a_ref[...], preferred_element_type=jnp.float32)
    z = z + jnp.dot(agg, w3b_ref[...], preferred_element_type=jnp.float32)
    z = jnp.maximum(z + b3_ref[...], 0.0)
    out_ref[...] = jnp.dot(z, w4_ref[...], preferred_element_type=jnp.float32) + b4_ref[...]


_sc_mesh = plsc.VectorSubcoreMesh(
    core_axis_name="c", subcore_axis_name="s", num_cores=NC, num_subcores=NS)


@functools.partial(
    pl.kernel,
    out_type=(jax.ShapeDtypeStruct((NC, N_PAD, HIDDEN), jnp.float32),
              jax.ShapeDtypeStruct((NC, DEG_ROWS, 16), jnp.float32)),
    mesh=_sc_mesh,
    scratch_types=[
        pltpu.VMEM((NB, BURST), jnp.int32),
        pltpu.VMEM((NB, BURST), jnp.int32),
        pltpu.VMEM((BURST, HIDDEN), jnp.float32),
        pltpu.VMEM((ROWS_PER_TILE, HIDDEN), jnp.float32),
        pltpu.VMEM((DEG_ROWS, 16), jnp.float32),
        pltpu.VMEM((5, BURST), jnp.int32),
        pltpu.VMEM((DEG_ROWS_PER_TILE, 16), jnp.float32),
        pltpu.VMEM_SHARED((N_PAD, HIDDEN), jnp.float32),
        pltpu.VMEM_SHARED((DEG_ROWS, 16), jnp.float32),
        pltpu.SemaphoreType.DMA,
    ],
    compiler_params=pltpu.CompilerParams(
        needs_layout_passes=False, use_tc_tiling_on_sc=False),
)
def _sc_agg(m_hbm, col_hbm, row_hbm, outa_hbm, outd_hbm,
            col_v, row_v, gbuf, wbuf, ldeg, didx, dbuf, agg_sh, deg_sh, sem):
    c = lax.axis_index("c")
    s = lax.axis_index("s")
    wid = s * NC + c

    pltpu.sync_copy(col_hbm.at[wid], col_v)
    pltpu.sync_copy(row_hbm.at[wid], row_v)

    zvec = jnp.zeros((16,), jnp.float32)
    ovec = jnp.ones((16,), jnp.float32)
    i16 = lax.iota(jnp.int32, 16)

    def _zero_wbuf(r, carry):
        for jj in range(HIDDEN // 16):
            wbuf[r, pl.ds(jj * 16, 16)] = zvec
        return carry

    lax.fori_loop(0, ROWS_PER_TILE, _zero_wbuf, 0)

    def _zero_ldeg(r, carry):
        ldeg[r, :] = zvec
        return carry

    lax.fori_loop(0, DEG_ROWS, _zero_ldeg, 0)
    for r in range(DEG_ROWS_PER_TILE):
        dbuf[r, :] = zvec
    for r in range(5):
        for k in range(BURST // 16):
            didx[r, pl.ds(k * 16, 16)] = i16 + (r * BURST + k * 16)

    r0 = s * ROWS_PER_TILE
    d0 = s * DEG_ROWS_PER_TILE
    pltpu.sync_copy(wbuf, agg_sh.at[pl.ds(r0, ROWS_PER_TILE)])
    pltpu.sync_copy(dbuf, deg_sh.at[pl.ds(d0, DEG_ROWS_PER_TILE)])
    plsc.subcore_barrier()
    pl.delay(SETTLE_NS)
    plsc.subcore_barrier()

    def _burst(j, carry):
        pltpu.async_copy(m_hbm.at[col_v.at[j]], gbuf, sem).wait()
        pltpu.sync_copy(gbuf, agg_sh.at[row_v.at[j]], add=True)
        for k in range(BURST // 16):
            rows16 = row_v[j, pl.ds(k * 16, 16)]
            rhi = lax.shift_right_logical(rows16, 4)
            rlo = jnp.bitwise_and(rows16, 15)
            plsc.addupdate_scatter(ldeg, [rhi, rlo], ovec)
        return carry

    lax.fori_loop(0, NB, _burst, 0)

    for r in range(5):
        pltpu.sync_copy(ldeg.at[pl.ds(r * BURST, BURST)],
                        deg_sh.at[didx.at[r]], add=True)

    plsc.subcore_barrier()
    pltpu.sync_copy(agg_sh.at[pl.ds(r0, ROWS_PER_TILE)], wbuf)
    pl.delay(SETTLE_NS)
    plsc.subcore_barrier()

    pltpu.sync_copy(agg_sh.at[pl.ds(r0, ROWS_PER_TILE)], wbuf)
    pltpu.sync_copy(wbuf, outa_hbm.at[c, pl.ds(r0, ROWS_PER_TILE)])
    pltpu.sync_copy(deg_sh.at[pl.ds(d0, DEG_ROWS_PER_TILE)], dbuf)
    pltpu.sync_copy(dbuf, outd_hbm.at[c, pl.ds(d0, DEG_ROWS_PER_TILE)])


def kernel(h, edge_index, W1, b1, W2, b2, W3, b3, W4, b4):
    ei = edge_index.astype(jnp.int32)
    pad = E_PAD - N_EDGES
    pad_i = jnp.arange(pad, dtype=jnp.int32)
    row = jnp.concatenate([ei[0], N_NODES + pad_i % (N_PAD - N_NODES)])
    col = jnp.concatenate([ei[1], pad_i % N_NODES])
    row3 = row.reshape(NW, NB, BURST)
    col3 = col.reshape(NW, NB, BURST)

    m_tab = pl.pallas_call(
        _msg_body,
        out_shape=jax.ShapeDtypeStruct((N_NODES, HIDDEN), jnp.float32),
    )(h, W1, b1.reshape(1, HIDDEN), W2, b2.reshape(1, HIDDEN))

    pagg, pdeg = _sc_agg(m_tab, col3, row3)
    deg2 = pdeg.reshape(NC, N_DEG)[:, :N_NODES].T

    out = pl.pallas_call(
        _upd_body,
        out_shape=jax.ShapeDtypeStruct((N_NODES, 2), jnp.float32),
    )(h, pagg, deg2, W3[:IN_DIM], W3[IN_DIM:], b3.reshape(1, HIDDEN),
      W4, b4.reshape(1, 2))
    return out


# speedup vs baseline: 10.0922x; 1.1639x over previous
import functools

import jax
import jax.numpy as jnp
from jax import lax
from jax.experimental import pallas as pl
from jax.experimental.pallas import tpu as pltpu
from jax.experimental.pallas import tpu_sc as plsc

N_NODES = 10000
N_EDGES = 320000
IN_DIM = 128
HIDDEN = 64

NC = 2
NS = 16
NW = NC * NS

BURST = 128
NB = 80
EDGES_PER_TILE = NB * BURST
E_PAD = NW * EDGES_PER_TILE
N_PAD = 10112
ROWS_PER_TILE = N_PAD // NS
DEG_ROWS = 640
N_DEG = DEG_ROWS * 16
DEG_ROWS_PER_TILE = DEG_ROWS // NS
SETTLE_NS = 3000


def _msg_body(h_ref, w1_ref, b1_ref, w2_ref, b2_ref, out_ref):
    m1 = jnp.dot(h_ref[...], w1_ref[...], preferred_element_type=jnp.float32)
    m1 = jnp.maximum(m1 + b1_ref[...], 0.0)
    m2 = jnp.dot(m1, w2_ref[...], preferred_element_type=jnp.float32)
    out_ref[...] = jnp.maximum(m2 + b2_ref[...], 0.0)


def _upd_body(h_ref, pa_ref, dd_ref, w3a_ref, w3b_ref, b3_ref, w4_ref, b4_ref,
              out_ref):
    p = pa_ref[0, :N_NODES, :] + pa_ref[1, :N_NODES, :]
    deg = jnp.maximum(dd_ref[:, 0:1] + dd_ref[:, 1:2], 1.0)
    agg = p / deg
    z = jnp.dot(h_ref[...], w3a_ref[...], preferred_element_type=jnp.float32)
    z = z + jnp.dot(agg, w3b_ref[...], preferred_element_type=jnp.float32)
    z = jnp.maximum(z + b3_ref[...], 0.0)
    out_ref[...] = jnp.dot(z, w4_ref[...], preferred_element_type=jnp.float32) + b4_ref[...]


_sc_mesh = plsc.VectorSubcoreMesh(
    core_axis_name="c", subcore_axis_name="s", num_cores=NC, num_subcores=NS)


@functools.partial(
    pl.kernel,
    out_type=(jax.ShapeDtypeStruct((NC, N_PAD, HIDDEN), jnp.float32),
              jax.ShapeDtypeStruct((NC, DEG_ROWS, 16), jnp.float32)),
    mesh=_sc_mesh,
    scratch_types=[
        pltpu.VMEM((NB, BURST), jnp.int32),
        pltpu.VMEM((NB, BURST), jnp.int32),
        pltpu.VMEM((BURST, HIDDEN), jnp.float32),
        pltpu.VMEM((BURST, HIDDEN), jnp.float32),
        pltpu.VMEM((ROWS_PER_TILE, HIDDEN), jnp.float32),
        pltpu.VMEM((DEG_ROWS, 16), jnp.float32),
        pltpu.VMEM((5, BURST), jnp.int32),
        pltpu.VMEM((DEG_ROWS_PER_TILE, 16), jnp.float32),
        pltpu.VMEM_SHARED((N_PAD, HIDDEN), jnp.float32),
        pltpu.VMEM_SHARED((DEG_ROWS, 16), jnp.float32),
        pltpu.SemaphoreType.DMA,
        pltpu.SemaphoreType.DMA,
        pltpu.SemaphoreType.DMA,
    ],
    compiler_params=pltpu.CompilerParams(
        needs_layout_passes=False, use_tc_tiling_on_sc=False),
)
def _sc_agg(m_hbm, col_hbm, row_hbm, outa_hbm, outd_hbm,
            col_v, row_v, gbuf0, gbuf1, wbuf, ldeg, didx, dbuf, agg_sh, deg_sh,
            sem0, sem1, sem2):
    c = lax.axis_index("c")
    s = lax.axis_index("s")
    wid = s * NC + c

    pltpu.sync_copy(col_hbm.at[wid], col_v)
    pltpu.sync_copy(row_hbm.at[wid], row_v)

    zvec = jnp.zeros((16,), jnp.float32)
    ovec = jnp.ones((16,), jnp.float32)
    i16 = lax.iota(jnp.int32, 16)

    def _zero_wbuf(r, carry):
        for jj in range(HIDDEN // 16):
            wbuf[r, pl.ds(jj * 16, 16)] = zvec
        return carry

    lax.fori_loop(0, ROWS_PER_TILE, _zero_wbuf, 0)

    def _zero_ldeg(r, carry):
        ldeg[r, :] = zvec
        return carry

    lax.fori_loop(0, DEG_ROWS, _zero_ldeg, 0)
    for r in range(DEG_ROWS_PER_TILE):
        dbuf[r, :] = zvec
    for r in range(5):
        for k in range(BURST // 16):
            didx[r, pl.ds(k * 16, 16)] = i16 + (r * BURST + k * 16)

    r0 = s * ROWS_PER_TILE
    d0 = s * DEG_ROWS_PER_TILE
    pltpu.sync_copy(wbuf, agg_sh.at[pl.ds(r0, ROWS_PER_TILE)])
    pltpu.sync_copy(dbuf, deg_sh.at[pl.ds(d0, DEG_ROWS_PER_TILE)])
    plsc.subcore_barrier()
    pl.delay(SETTLE_NS)
    plsc.subcore_barrier()

    def _deg_hist(j):
        for k in range(BURST // 16):
            rows16 = row_v[j, pl.ds(k * 16, 16)]
            rhi = lax.shift_right_logical(rows16, 4)
            rlo = jnp.bitwise_and(rows16, 15)
            plsc.addupdate_scatter(ldeg, [rhi, rlo], ovec)

    pltpu.async_copy(m_hbm.at[col_v.at[0]], gbuf0, sem0)

    def _pair(i, carry):
        j0 = 2 * i
        j1 = j0 + 1
        pltpu.make_async_copy(m_hbm.at[col_v.at[j0]], gbuf0, sem0).wait()
        pltpu.async_copy(m_hbm.at[col_v.at[j1]], gbuf1, sem1)
        sc0 = pltpu.async_copy(gbuf0, agg_sh.at[row_v.at[j0]], sem2, add=True)
        _deg_hist(j0)
        sc0.wait()
        pltpu.make_async_copy(m_hbm.at[col_v.at[j1]], gbuf1, sem1).wait()

        @pl.when(i < NB // 2 - 1)
        def _next():
            pltpu.async_copy(m_hbm.at[col_v.at[j0 + 2]], gbuf0, sem0)

        sc1 = pltpu.async_copy(gbuf1, agg_sh.at[row_v.at[j1]], sem2, add=True)
        _deg_hist(j1)
        sc1.wait()
        return carry

    lax.fori_loop(0, NB // 2, _pair, 0)

    for r in range(5):
        pltpu.sync_copy(ldeg.at[pl.ds(r * BURST, BURST)],
                        deg_sh.at[didx.at[r]], add=True)

    plsc.subcore_barrier()
    pltpu.sync_copy(agg_sh.at[pl.ds(r0, ROWS_PER_TILE)], wbuf)
    pl.delay(SETTLE_NS)
    plsc.subcore_barrier()

    pltpu.sync_copy(agg_sh.at[pl.ds(r0, ROWS_PER_TILE)], wbuf)
    pltpu.sync_copy(wbuf, outa_hbm.at[c, pl.ds(r0, ROWS_PER_TILE)])
    pltpu.sync_copy(deg_sh.at[pl.ds(d0, DEG_ROWS_PER_TILE)], dbuf)
    pltpu.sync_copy(dbuf, outd_hbm.at[c, pl.ds(d0, DEG_ROWS_PER_TILE)])


def kernel(h, edge_index, W1, b1, W2, b2, W3, b3, W4, b4):
    ei = edge_index.astype(jnp.int32)
    pad = E_PAD - N_EDGES
    pad_i = jnp.arange(pad, dtype=jnp.int32)
    row = jnp.concatenate([ei[0], N_NODES + pad_i % (N_PAD - N_NODES)])
    col = jnp.concatenate([ei[1], pad_i % N_NODES])
    row3 = row.reshape(NW, NB, BURST)
    col3 = col.reshape(NW, NB, BURST)

    m_tab = pl.pallas_call(
        _msg_body,
        out_shape=jax.ShapeDtypeStruct((N_NODES, HIDDEN), jnp.float32),
    )(h, W1, b1.reshape(1, HIDDEN), W2, b2.reshape(1, HIDDEN))

    pagg, pdeg = _sc_agg(m_tab, col3, row3)
    deg2 = pdeg.reshape(NC, N_DEG)[:, :N_NODES].T

    out = pl.pallas_call(
        _upd_body,
        out_shape=jax.ShapeDtypeStruct((N_NODES, 2), jnp.float32),
    )(h, pagg, deg2, W3[:IN_DIM], W3[IN_DIM:], b3.reshape(1, HIDDEN),
      W4, b4.reshape(1, 2))
    return out


# speedup vs baseline: 12.8907x; 1.2773x over previous
import functools

import numpy as np

import jax
import jax.numpy as jnp
from jax import lax
from jax.experimental import pallas as pl
from jax.experimental.pallas import tpu as pltpu
from jax.experimental.pallas import tpu_sc as plsc

N_NODES = 10000
N_EDGES = 320000
IN_DIM = 128
HIDDEN = 64

NC = 2
NS = 16
NW = NC * NS

BURST = 128
NBUF = 4
NB = 80
EDGES_PER_TILE = NB * BURST
E_PAD = NW * EDGES_PER_TILE
N_PAD = 10112
ROWS_PER_TILE = N_PAD // NS
DEG_ROWS = 640
N_DEG = DEG_ROWS * 16
DEG_ROWS_PER_TILE = DEG_ROWS // NS
SETTLE_NS = 3000

_PAD_N = E_PAD - N_EDGES
_PAD_ROW = (N_NODES + np.arange(_PAD_N) % (N_PAD - N_NODES)).astype(np.int32)
_PAD_COL = (np.arange(_PAD_N) % N_NODES).astype(np.int32)


def _msg_body(h_ref, w1_ref, b1_ref, w2_ref, b2_ref, out_ref):
    m1 = jnp.dot(h_ref[...], w1_ref[...], preferred_element_type=jnp.float32)
    m1 = jnp.maximum(m1 + b1_ref[...], 0.0)
    m2 = jnp.dot(m1, w2_ref[...], preferred_element_type=jnp.float32)
    out_ref[...] = jnp.maximum(m2 + b2_ref[...], 0.0)


def _upd_body(h_ref, pa_ref, d0_ref, d1_ref, w3a_ref, w3b_ref, b3_ref, w4_ref,
              b4_ref, out_ref):
    p = pa_ref[0, :N_NODES, :] + pa_ref[1, :N_NODES, :]
    deg = jnp.maximum(d0_ref[...] + d1_ref[...], 1.0)
    agg = p / deg
    z = jnp.dot(h_ref[...], w3a_ref[...], preferred_element_type=jnp.float32)
    z = z + jnp.dot(agg, w3b_ref[...], preferred_element_type=jnp.float32)
    z = jnp.maximum(z + b3_ref[...], 0.0)
    out_ref[...] = jnp.dot(z, w4_ref[...], preferred_element_type=jnp.float32) + b4_ref[...]


_sc_mesh = plsc.VectorSubcoreMesh(
    core_axis_name="c", subcore_axis_name="s", num_cores=NC, num_subcores=NS)


@functools.partial(
    pl.kernel,
    out_type=(jax.ShapeDtypeStruct((NC, N_PAD, HIDDEN), jnp.float32),
              jax.ShapeDtypeStruct((NC, DEG_ROWS, 16), jnp.float32)),
    mesh=_sc_mesh,
    scratch_types=[
        pltpu.VMEM((NB, BURST), jnp.int32),
        pltpu.VMEM((NB, BURST), jnp.int32),
        [pltpu.VMEM((BURST, HIDDEN), jnp.float32) for _ in range(NBUF)],
        pltpu.VMEM((DEG_ROWS, 16), jnp.float32),
        pltpu.VMEM((5, BURST), jnp.int32),
        pltpu.VMEM((DEG_ROWS_PER_TILE, 16), jnp.float32),
        pltpu.VMEM_SHARED((N_PAD, HIDDEN), jnp.float32),
        pltpu.VMEM_SHARED((DEG_ROWS, 16), jnp.float32),
        [pltpu.SemaphoreType.DMA for _ in range(NBUF)],
        [pltpu.SemaphoreType.DMA for _ in range(NBUF)],
    ],
    compiler_params=pltpu.CompilerParams(
        needs_layout_passes=False, use_tc_tiling_on_sc=False),
)
def _sc_agg(m_hbm, col_hbm, row_hbm, outa_hbm, outd_hbm,
            col_v, row_v, gbufs, ldeg, didx, dbuf, agg_sh, deg_sh, gsem, ssem):
    c = lax.axis_index("c")
    s = lax.axis_index("s")
    wid = s * NC + c

    pltpu.sync_copy(col_hbm.at[wid], col_v)
    pltpu.sync_copy(row_hbm.at[wid], row_v)

    zvec = jnp.zeros((16,), jnp.float32)
    ovec = jnp.ones((16,), jnp.float32)
    i16 = lax.iota(jnp.int32, 16)

    def _zero_gbuf0(r, carry):
        for jj in range(HIDDEN // 16):
            gbufs[0][r, pl.ds(jj * 16, 16)] = zvec
        return carry

    lax.fori_loop(0, BURST, _zero_gbuf0, 0)

    def _zero_ldeg(r, carry):
        ldeg[r, :] = zvec
        return carry

    lax.fori_loop(0, DEG_ROWS, _zero_ldeg, 0)
    for r in range(DEG_ROWS_PER_TILE):
        dbuf[r, :] = zvec
    for r in range(5):
        for k in range(BURST // 16):
            didx[r, pl.ds(k * 16, 16)] = i16 + (r * BURST + k * 16)

    r0 = s * ROWS_PER_TILE
    d0 = s * DEG_ROWS_PER_TILE
    for t in range(4):
        pltpu.sync_copy(gbufs[0], agg_sh.at[pl.ds(r0 + t * BURST, BURST)])
    pltpu.sync_copy(gbufs[0].at[pl.ds(0, ROWS_PER_TILE - 4 * BURST)],
                    agg_sh.at[pl.ds(r0 + 4 * BURST, ROWS_PER_TILE - 4 * BURST)])
    pltpu.sync_copy(dbuf, deg_sh.at[pl.ds(d0, DEG_ROWS_PER_TILE)])
    plsc.subcore_barrier()
    pl.delay(SETTLE_NS)
    plsc.subcore_barrier()

    def _deg_hist(j):
        for k in range(BURST // 16):
            rows16 = row_v[j, pl.ds(k * 16, 16)]
            rhi = lax.shift_right_logical(rows16, 4)
            rlo = jnp.bitwise_and(rows16, 15)
            plsc.addupdate_scatter(ldeg, [rhi, rlo], ovec)

    for b in range(NBUF):
        pltpu.async_copy(m_hbm.at[col_v.at[b]], gbufs[b], gsem[b])

    def _octet(i, carry):
        for b in range(NBUF):
            j = i * NBUF + b
            bp = (b - 1) % NBUF
            pltpu.make_async_copy(m_hbm.at[col_v.at[j]], gbufs[b], gsem[b]).wait()
            pltpu.async_copy(gbufs[b], agg_sh.at[row_v.at[j]], ssem[b], add=True)
            _deg_hist(j)
            jm1 = jnp.maximum(j - 1, 0)
            if b == 0:
                @pl.when(j >= 1)
                def _wait_prev():
                    pltpu.make_async_copy(
                        gbufs[bp], agg_sh.at[row_v.at[jm1]], ssem[bp]).wait()
            else:
                pltpu.make_async_copy(
                    gbufs[bp], agg_sh.at[row_v.at[jm1]], ssem[bp]).wait()

            @pl.when(j + NBUF - 1 < NB)
            def _issue_next():
                pltpu.async_copy(
                    m_hbm.at[col_v.at[j + NBUF - 1]], gbufs[bp], gsem[bp])
        return carry

    lax.fori_loop(0, NB // NBUF, _octet, 0)
    pltpu.make_async_copy(
        gbufs[(NB - 1) % NBUF], agg_sh.at[row_v.at[NB - 1]],
        ssem[(NB - 1) % NBUF]).wait()

    for r in range(5):
        pltpu.sync_copy(ldeg.at[pl.ds(r * BURST, BURST)],
                        deg_sh.at[didx.at[r]], add=True)

    plsc.subcore_barrier()
    pltpu.sync_copy(agg_sh.at[pl.ds(r0, BURST)], gbufs[1])
    pl.delay(SETTLE_NS)
    plsc.subcore_barrier()

    for t in range(4):
        pltpu.sync_copy(agg_sh.at[pl.ds(r0 + t * BURST, BURST)], gbufs[t % 2])
        pltpu.sync_copy(gbufs[t % 2], outa_hbm.at[c, pl.ds(r0 + t * BURST, BURST)])
    tail = ROWS_PER_TILE - 4 * BURST
    pltpu.sync_copy(agg_sh.at[pl.ds(r0 + 4 * BURST, tail)],
                    gbufs[2].at[pl.ds(0, tail)])
    pltpu.sync_copy(gbufs[2].at[pl.ds(0, tail)],
                    outa_hbm.at[c, pl.ds(r0 + 4 * BURST, tail)])
    pltpu.sync_copy(deg_sh.at[pl.ds(d0, DEG_ROWS_PER_TILE)], dbuf)
    pltpu.sync_copy(dbuf, outd_hbm.at[c, pl.ds(d0, DEG_ROWS_PER_TILE)])


def kernel(h, edge_index, W1, b1, W2, b2, W3, b3, W4, b4):
    ei = edge_index.astype(jnp.int32)
    row = jnp.concatenate([ei[0], jnp.asarray(_PAD_ROW)])
    col = jnp.concatenate([ei[1], jnp.asarray(_PAD_COL)])
    row3 = row.reshape(NW, NB, BURST)
    col3 = col.reshape(NW, NB, BURST)

    m_tab = pl.pallas_call(
        _msg_body,
        out_shape=jax.ShapeDtypeStruct((N_NODES, HIDDEN), jnp.float32),
    )(h, W1, b1.reshape(1, HIDDEN), W2, b2.reshape(1, HIDDEN))

    pagg, pdeg = _sc_agg(m_tab, col3, row3)
    degflat = pdeg.reshape(NC, N_DEG)
    dd0 = degflat[0, :N_NODES, None]
    dd1 = degflat[1, :N_NODES, None]

    out = pl.pallas_call(
        _upd_body,
        out_shape=jax.ShapeDtypeStruct((N_NODES, 2), jnp.float32),
    )(h, pagg, dd0, dd1, W3[:IN_DIM], W3[IN_DIM:], b3.reshape(1, HIDDEN),
      W4, b4.reshape(1, 2))
    return out


# speedup vs baseline: 13.0665x; 1.0136x over previous
import functools

import numpy as np

import jax
import jax.numpy as jnp
from jax import lax
from jax.experimental import pallas as pl
from jax.experimental.pallas import tpu as pltpu
from jax.experimental.pallas import tpu_sc as plsc

N_NODES = 10000
N_EDGES = 320000
IN_DIM = 128
HIDDEN = 64

NC = 2
NS = 16
NW = NC * NS

BURST = 128
NBUF = 4
NB = 80
EDGES_PER_TILE = NB * BURST
E_PAD = NW * EDGES_PER_TILE
N_PAD = 10112
ROWS_PER_TILE = N_PAD // NS
DEG_ROWS = 640
N_DEG = DEG_ROWS * 16
DEG_ROWS_PER_TILE = DEG_ROWS // NS
SETTLE_NS = 3000

_PAD_N = E_PAD - N_EDGES
_PAD_ROW = (N_NODES + np.arange(_PAD_N) % (N_PAD - N_NODES)).astype(np.int32)
_PAD_COL = (np.arange(_PAD_N) % N_NODES).astype(np.int32)


def _msg_body(h_ref, w1_ref, b1_ref, w2_ref, b2_ref, out_ref):
    m1 = jnp.dot(h_ref[...], w1_ref[...], preferred_element_type=jnp.float32)
    m1 = jnp.maximum(m1 + b1_ref[...], 0.0)
    m2 = jnp.dot(m1, w2_ref[...], preferred_element_type=jnp.float32)
    out_ref[...] = jnp.maximum(m2 + b2_ref[...], 0.0)


def _upd_body(h_ref, pa_ref, d0_ref, d1_ref, w3a_ref, w3b_ref, b3_ref, w4_ref,
              b4_ref, out_ref):
    p = pa_ref[0, :N_NODES, :] + pa_ref[1, :N_NODES, :]
    deg = jnp.maximum(d0_ref[...] + d1_ref[...], 1.0)
    agg = p / deg
    z = jnp.dot(h_ref[...], w3a_ref[...], preferred_element_type=jnp.float32)
    z = z + jnp.dot(agg, w3b_ref[...], preferred_element_type=jnp.float32)
    z = jnp.maximum(z + b3_ref[...], 0.0)
    out_ref[...] = jnp.dot(z, w4_ref[...], preferred_element_type=jnp.float32) + b4_ref[...]


_sc_mesh = plsc.VectorSubcoreMesh(
    core_axis_name="c", subcore_axis_name="s", num_cores=NC, num_subcores=NS)


@functools.partial(
    pl.kernel,
    out_type=(jax.ShapeDtypeStruct((NC, N_PAD, HIDDEN), jnp.float32),
              jax.ShapeDtypeStruct((NC, DEG_ROWS, 16), jnp.float32)),
    mesh=_sc_mesh,
    scratch_types=[
        pltpu.VMEM((NB, BURST), jnp.int32),
        pltpu.VMEM((NB, BURST), jnp.int32),
        [pltpu.VMEM((BURST, HIDDEN), jnp.float32) for _ in range(NBUF)],
        pltpu.VMEM((DEG_ROWS, 16), jnp.float32),
        pltpu.VMEM((5, BURST), jnp.int32),
        pltpu.VMEM((DEG_ROWS_PER_TILE, 16), jnp.float32),
        pltpu.VMEM_SHARED((N_PAD, HIDDEN), jnp.float32),
        pltpu.VMEM_SHARED((DEG_ROWS, 16), jnp.float32),
        [pltpu.SemaphoreType.DMA for _ in range(NBUF)],
        [pltpu.SemaphoreType.DMA for _ in range(NBUF)],
    ],
    compiler_params=pltpu.CompilerParams(
        needs_layout_passes=False, use_tc_tiling_on_sc=False),
)
def _sc_agg(m_hbm, col_hbm, row_hbm, outa_hbm, outd_hbm,
            col_v, row_v, gbufs, ldeg, didx, dbuf, agg_sh, deg_sh, gsem, ssem):
    c = lax.axis_index("c")
    s = lax.axis_index("s")
    wid = s * NC + c

    pltpu.sync_copy(col_hbm.at[wid], col_v)
    pltpu.sync_copy(row_hbm.at[wid], row_v)

    zvec = jnp.zeros((16,), jnp.float32)
    ovec = jnp.ones((16,), jnp.float32)
    i16 = lax.iota(jnp.int32, 16)

    def _zero_gbuf0(r, carry):
        for jj in range(HIDDEN // 16):
            gbufs[0][r, pl.ds(jj * 16, 16)] = zvec
        return carry

    lax.fori_loop(0, BURST, _zero_gbuf0, 0)

    def _zero_ldeg(r, carry):
        ldeg[r, :] = zvec
        return carry

    lax.fori_loop(0, DEG_ROWS, _zero_ldeg, 0)
    for r in range(DEG_ROWS_PER_TILE):
        dbuf[r, :] = zvec
    for r in range(5):
        for k in range(BURST // 16):
            didx[r, pl.ds(k * 16, 16)] = i16 + (r * BURST + k * 16)

    r0 = s * ROWS_PER_TILE
    d0 = s * DEG_ROWS_PER_TILE
    for t in range(4):
        pltpu.sync_copy(gbufs[0], agg_sh.at[pl.ds(r0 + t * BURST, BURST)])
    pltpu.sync_copy(gbufs[0].at[pl.ds(0, ROWS_PER_TILE - 4 * BURST)],
                    agg_sh.at[pl.ds(r0 + 4 * BURST, ROWS_PER_TILE - 4 * BURST)])
    pltpu.sync_copy(dbuf, deg_sh.at[pl.ds(d0, DEG_ROWS_PER_TILE)])
    plsc.subcore_barrier()
    pl.delay(SETTLE_NS)
    plsc.subcore_barrier()

    def _deg_hist(j):
        for k in range(BURST // 16):
            rows16 = row_v[j, pl.ds(k * 16, 16)]
            rhi = lax.shift_right_logical(rows16, 4)
            rlo = jnp.bitwise_and(rows16, 15)
            plsc.addupdate_scatter(ldeg, [rhi, rlo], ovec)

    for b in range(NBUF):
        pltpu.async_copy(m_hbm.at[col_v.at[b]], gbufs[b], gsem[b])

    def _octet(i, carry):
        for b in range(NBUF):
            j = i * NBUF + b
            bp = (b - 1) % NBUF
            pltpu.make_async_copy(m_hbm.at[col_v.at[j]], gbufs[b], gsem[b]).wait()
            pltpu.async_copy(gbufs[b], agg_sh.at[row_v.at[j]], ssem[b], add=True)
            _deg_hist(j)
            jm1 = jnp.maximum(j - 1, 0)
            if b == 0:
                @pl.when(j >= 1)
                def _wait_prev():
                    pltpu.make_async_copy(
                        gbufs[bp], agg_sh.at[row_v.at[jm1]], ssem[bp]).wait()
            else:
                pltpu.make_async_copy(
                    gbufs[bp], agg_sh.at[row_v.at[jm1]], ssem[bp]).wait()

            @pl.when(jnp.logical_and(j >= 1, j + NBUF - 1 < NB))
            def _issue_next():
                pltpu.async_copy(
                    m_hbm.at[col_v.at[j + NBUF - 1]], gbufs[bp], gsem[bp])
        return carry

    lax.fori_loop(0, NB // NBUF, _octet, 0)
    pltpu.make_async_copy(
        gbufs[(NB - 1) % NBUF], agg_sh.at[row_v.at[NB - 1]],
        ssem[(NB - 1) % NBUF]).wait()

    for r in range(5):
        pltpu.sync_copy(ldeg.at[pl.ds(r * BURST, BURST)],
                        deg_sh.at[didx.at[r]], add=True)

    plsc.subcore_barrier()
    pltpu.sync_copy(agg_sh.at[pl.ds(r0, BURST)], gbufs[1])
    pl.delay(SETTLE_NS)
    plsc.subcore_barrier()

    for t in range(4):
        pltpu.sync_copy(agg_sh.at[pl.ds(r0 + t * BURST, BURST)], gbufs[t % 2])
        pltpu.sync_copy(gbufs[t % 2], outa_hbm.at[c, pl.ds(r0 + t * BURST, BURST)])
    tail = ROWS_PER_TILE - 4 * BURST
    pltpu.sync_copy(agg_sh.at[pl.ds(r0 + 4 * BURST, tail)],
                    gbufs[2].at[pl.ds(0, tail)])
    pltpu.sync_copy(gbufs[2].at[pl.ds(0, tail)],
                    outa_hbm.at[c, pl.ds(r0 + 4 * BURST, tail)])
    pltpu.sync_copy(deg_sh.at[pl.ds(d0, DEG_ROWS_PER_TILE)], dbuf)
    pltpu.sync_copy(dbuf, outd_hbm.at[c, pl.ds(d0, DEG_ROWS_PER_TILE)])


def kernel(h, edge_index, W1, b1, W2, b2, W3, b3, W4, b4):
    ei = edge_index.astype(jnp.int32)
    row = jnp.concatenate([ei[0], jnp.asarray(_PAD_ROW)])
    col = jnp.concatenate([ei[1], jnp.asarray(_PAD_COL)])
    row3 = row.reshape(NW, NB, BURST)
    col3 = col.reshape(NW, NB, BURST)

    m_tab = pl.pallas_call(
        _msg_body,
        out_shape=jax.ShapeDtypeStruct((N_NODES, HIDDEN), jnp.float32),
    )(h, W1, b1.reshape(1, HIDDEN), W2, b2.reshape(1, HIDDEN))

    pagg, pdeg = _sc_agg(m_tab, col3, row3)
    degflat = pdeg.reshape(NC, N_DEG)
    dd0 = degflat[0, :N_NODES, None]
    dd1 = degflat[1, :N_NODES, None]

    out = pl.pallas_call(
        _upd_body,
        out_shape=jax.ShapeDtypeStruct((N_NODES, 2), jnp.float32),
    )(h, pagg, dd0, dd1, W3[:IN_DIM], W3[IN_DIM:], b3.reshape(1, HIDDEN),
      W4, b4.reshape(1, 2))
    return out


# speedup vs baseline: 14.7363x; 1.1278x over previous
import functools

import jax
import jax.numpy as jnp
from jax import lax
from jax.experimental import pallas as pl
from jax.experimental.pallas import tpu as pltpu
from jax.experimental.pallas import tpu_sc as plsc

N_NODES = 10000
N_EDGES = 320000
IN_DIM = 128
HIDDEN = 64

NC = 2
NS = 16
NW = NC * NS

BURST = 128
NBUF = 4
NB = 80
EDGES_PER_TILE = NB * BURST
E_PAD = NW * EDGES_PER_TILE
N_PAD = 10112
ROWS_PER_TILE = N_PAD // NS
DEG_ROWS = 640
N_DEG = DEG_ROWS * 16
DEG_ROWS_PER_TILE = DEG_ROWS // NS
SETTLE_NS = 3000

EI_ROWS = N_EDGES // BURST
REAL_TAIL = EI_ROWS - (NW - 1) * NB


def _msg_body(h_ref, w1_ref, b1_ref, w2_ref, b2_ref, out_ref):
    m1 = jnp.dot(h_ref[...], w1_ref[...], preferred_element_type=jnp.float32)
    m1 = jnp.maximum(m1 + b1_ref[...], 0.0)
    m2 = jnp.dot(m1, w2_ref[...], preferred_element_type=jnp.float32)
    out_ref[...] = jnp.maximum(m2 + b2_ref[...], 0.0)


def _upd_body(h_ref, pa_ref, d0_ref, d1_ref, w3a_ref, w3b_ref, b3_ref, w4_ref,
              b4_ref, out_ref):
    p = pa_ref[0, :N_NODES, :] + pa_ref[1, :N_NODES, :]
    deg = jnp.maximum(d0_ref[...] + d1_ref[...], 1.0)
    agg = p / deg
    z = jnp.dot(h_ref[...], w3a_ref[...], preferred_element_type=jnp.float32)
    z = z + jnp.dot(agg, w3b_ref[...], preferred_element_type=jnp.float32)
    z = jnp.maximum(z + b3_ref[...], 0.0)
    out_ref[...] = jnp.dot(z, w4_ref[...], preferred_element_type=jnp.float32) + b4_ref[...]


_sc_mesh = plsc.VectorSubcoreMesh(
    core_axis_name="c", subcore_axis_name="s", num_cores=NC, num_subcores=NS)


@functools.partial(
    pl.kernel,
    out_type=(jax.ShapeDtypeStruct((NC, N_PAD, HIDDEN), jnp.float32),
              jax.ShapeDtypeStruct((NC, DEG_ROWS, 16), jnp.float32)),
    mesh=_sc_mesh,
    scratch_types=[
        pltpu.VMEM((NB, 2, BURST), jnp.int32),
        [pltpu.VMEM((BURST, HIDDEN), jnp.float32) for _ in range(NBUF)],
        pltpu.VMEM((DEG_ROWS, 16), jnp.float32),
        pltpu.VMEM((5, BURST), jnp.int32),
        pltpu.VMEM((DEG_ROWS_PER_TILE, 16), jnp.float32),
        pltpu.VMEM_SHARED((N_PAD, HIDDEN), jnp.float32),
        pltpu.VMEM_SHARED((DEG_ROWS, 16), jnp.float32),
        [pltpu.SemaphoreType.DMA for _ in range(NBUF)],
        [pltpu.SemaphoreType.DMA for _ in range(NBUF)],
    ],
    compiler_params=pltpu.CompilerParams(
        needs_layout_passes=False, use_tc_tiling_on_sc=False),
)
def _sc_agg(m_hbm, ei_hbm, outa_hbm, outd_hbm,
            ev_v, gbufs, ldeg, didx, dbuf, agg_sh, deg_sh, gsem, ssem):
    c = lax.axis_index("c")
    s = lax.axis_index("s")
    wid = s * NC + c

    zvec = jnp.zeros((16,), jnp.float32)
    ovec = jnp.ones((16,), jnp.float32)
    i16 = lax.iota(jnp.int32, 16)

    @pl.when(wid < NW - 1)
    def _load_full():
        pltpu.sync_copy(ei_hbm.at[pl.ds(wid * NB, NB)], ev_v)

    @pl.when(wid == NW - 1)
    def _load_tail():
        pltpu.sync_copy(ei_hbm.at[pl.ds((NW - 1) * NB, REAL_TAIL)],
                        ev_v.at[pl.ds(0, REAL_TAIL)])

        def _fill_pad(j, carry):
            base = j * BURST
            for k in range(BURST // 16):
                off = i16 + (base + k * 16)
                ev_v[j, 0, pl.ds(k * 16, 16)] = (
                    N_NODES + lax.rem(off, N_PAD - N_NODES))
                ev_v[j, 1, pl.ds(k * 16, 16)] = lax.rem(off, N_NODES)
            return carry

        lax.fori_loop(REAL_TAIL, NB, _fill_pad, 0)

    def _zero_gbuf0(r, carry):
        for jj in range(HIDDEN // 16):
            gbufs[0][r, pl.ds(jj * 16, 16)] = zvec
        return carry

    lax.fori_loop(0, BURST, _zero_gbuf0, 0)

    def _zero_ldeg(r, carry):
        ldeg[r, :] = zvec
        return carry

    lax.fori_loop(0, DEG_ROWS, _zero_ldeg, 0)
    for r in range(DEG_ROWS_PER_TILE):
        dbuf[r, :] = zvec
    for r in range(5):
        for k in range(BURST // 16):
            didx[r, pl.ds(k * 16, 16)] = i16 + (r * BURST + k * 16)

    r0 = s * ROWS_PER_TILE
    d0 = s * DEG_ROWS_PER_TILE
    for t in range(4):
        pltpu.sync_copy(gbufs[0], agg_sh.at[pl.ds(r0 + t * BURST, BURST)])
    pltpu.sync_copy(gbufs[0].at[pl.ds(0, ROWS_PER_TILE - 4 * BURST)],
                    agg_sh.at[pl.ds(r0 + 4 * BURST, ROWS_PER_TILE - 4 * BURST)])
    pltpu.sync_copy(dbuf, deg_sh.at[pl.ds(d0, DEG_ROWS_PER_TILE)])
    plsc.subcore_barrier()
    pl.delay(SETTLE_NS)
    plsc.subcore_barrier()

    def _deg_hist(j):
        for k in range(BURST // 16):
            rows16 = ev_v[j, 0, pl.ds(k * 16, 16)]
            rhi = lax.shift_right_logical(rows16, 4)
            rlo = jnp.bitwise_and(rows16, 15)
            plsc.addupdate_scatter(ldeg, [rhi, rlo], ovec)

    for b in range(NBUF):
        pltpu.async_copy(m_hbm.at[ev_v.at[b, 1]], gbufs[b], gsem[b])

    def _octet(i, carry):
        for b in range(NBUF):
            j = i * NBUF + b
            bp = (b - 1) % NBUF
            pltpu.make_async_copy(m_hbm.at[ev_v.at[j, 1]], gbufs[b], gsem[b]).wait()
            pltpu.async_copy(gbufs[b], agg_sh.at[ev_v.at[j, 0]], ssem[b], add=True)
            _deg_hist(j)
            jm1 = jnp.maximum(j - 1, 0)
            if b == 0:
                @pl.when(j >= 1)
                def _wait_prev():
                    pltpu.make_async_copy(
                        gbufs[bp], agg_sh.at[ev_v.at[jm1, 0]], ssem[bp]).wait()
            else:
                pltpu.make_async_copy(
                    gbufs[bp], agg_sh.at[ev_v.at[jm1, 0]], ssem[bp]).wait()

            @pl.when(jnp.logical_and(j >= 1, j + NBUF - 1 < NB))
            def _issue_next():
                pltpu.async_copy(
                    m_hbm.at[ev_v.at[j + NBUF - 1, 1]], gbufs[bp], gsem[bp])
        return carry

    lax.fori_loop(0, NB // NBUF, _octet, 0)
    pltpu.make_async_copy(
        gbufs[(NB - 1) % NBUF], agg_sh.at[ev_v.at[NB - 1, 0]],
        ssem[(NB - 1) % NBUF]).wait()

    for r in range(5):
        pltpu.sync_copy(ldeg.at[pl.ds(r * BURST, BURST)],
                        deg_sh.at[didx.at[r]], add=True)

    plsc.subcore_barrier()
    pltpu.sync_copy(agg_sh.at[pl.ds(r0, BURST)], gbufs[1])
    pl.delay(SETTLE_NS)
    plsc.subcore_barrier()

    for t in range(4):
        pltpu.sync_copy(agg_sh.at[pl.ds(r0 + t * BURST, BURST)], gbufs[t % 2])
        pltpu.sync_copy(gbufs[t % 2], outa_hbm.at[c, pl.ds(r0 + t * BURST, BURST)])
    tail = ROWS_PER_TILE - 4 * BURST
    pltpu.sync_copy(agg_sh.at[pl.ds(r0 + 4 * BURST, tail)],
                    gbufs[2].at[pl.ds(0, tail)])
    pltpu.sync_copy(gbufs[2].at[pl.ds(0, tail)],
                    outa_hbm.at[c, pl.ds(r0 + 4 * BURST, tail)])
    pltpu.sync_copy(deg_sh.at[pl.ds(d0, DEG_ROWS_PER_TILE)], dbuf)
    pltpu.sync_copy(dbuf, outd_hbm.at[c, pl.ds(d0, DEG_ROWS_PER_TILE)])


def kernel(h, edge_index, W1, b1, W2, b2, W3, b3, W4, b4):
    ei = edge_index.astype(jnp.int32)
    ei3 = ei.reshape(2, EI_ROWS, BURST).transpose(1, 0, 2)

    m_tab = pl.pallas_call(
        _msg_body,
        out_shape=jax.ShapeDtypeStruct((N_NODES, HIDDEN), jnp.float32),
    )(h, W1, b1.reshape(1, HIDDEN), W2, b2.reshape(1, HIDDEN))

    pagg, pdeg = _sc_agg(m_tab, ei3)
    degflat = pdeg.reshape(NC, N_DEG)
    dd0 = degflat[0, :N_NODES, None]
    dd1 = degflat[1, :N_NODES, None]

    out = pl.pallas_call(
        _upd_body,
        out_shape=jax.ShapeDtypeStruct((N_NODES, 2), jnp.float32),
    )(h, pagg, dd0, dd1, W3[:IN_DIM], W3[IN_DIM:], b3.reshape(1, HIDDEN),
      W4, b4.reshape(1, 2))
    return out


# speedup vs baseline: 15.0389x; 1.0205x over previous
import functools

import jax
import jax.numpy as jnp
from jax import lax
from jax.experimental import pallas as pl
from jax.experimental.pallas import tpu as pltpu
from jax.experimental.pallas import tpu_sc as plsc

N_NODES = 10000
N_EDGES = 320000
IN_DIM = 128
HIDDEN = 64

NC = 2
NS = 16
NW = NC * NS

BURST = 128
NBUF = 4
NB = 80
EDGES_PER_TILE = NB * BURST
E_PAD = NW * EDGES_PER_TILE
N_PAD = 10112
ROWS_PER_TILE = N_PAD // NS
DEG_ROWS = 640
N_DEG = DEG_ROWS * 16
DEG_ROWS_PER_TILE = DEG_ROWS // NS
SETTLE_NS = 3000

EI_ROWS = N_EDGES // BURST
REAL_TAIL = EI_ROWS - (NW - 1) * NB


def _msg_body(h_ref, w1_ref, b1_ref, w2_ref, b2_ref, out_ref):
    m1 = jnp.dot(h_ref[...], w1_ref[...], preferred_element_type=jnp.float32)
    m1 = jnp.maximum(m1 + b1_ref[...], 0.0)
    m2 = jnp.dot(m1, w2_ref[...], preferred_element_type=jnp.float32)
    out_ref[...] = jnp.maximum(m2 + b2_ref[...], 0.0)


def _upd_body(h_ref, pa_ref, d0_ref, d1_ref, w3a_ref, w3b_ref, b3_ref, w4_ref,
              b4_ref, out_ref):
    p = (pa_ref[0, :N_NODES, :HIDDEN] + pa_ref[1, :N_NODES, :HIDDEN])
    deg = jnp.maximum(d0_ref[...] + d1_ref[...], 1.0)
    agg = p / deg
    z = jnp.dot(h_ref[...], w3a_ref[...], preferred_element_type=jnp.float32)
    z = z + jnp.dot(agg, w3b_ref[...], preferred_element_type=jnp.float32)
    z = jnp.maximum(z + b3_ref[...], 0.0)
    out_ref[...] = jnp.dot(z, w4_ref[...], preferred_element_type=jnp.float32) + b4_ref[...]


_sc_mesh = plsc.VectorSubcoreMesh(
    core_axis_name="c", subcore_axis_name="s", num_cores=NC, num_subcores=NS)


@functools.partial(
    pl.kernel,
    out_type=(jax.ShapeDtypeStruct((NC, N_PAD, 128), jnp.float32),
              jax.ShapeDtypeStruct((NC, DEG_ROWS, 16), jnp.float32)),
    mesh=_sc_mesh,
    scratch_types=[
        pltpu.VMEM((NB, 2, BURST), jnp.int32),
        [pltpu.VMEM((BURST, HIDDEN), jnp.float32) for _ in range(NBUF)],
        pltpu.VMEM((DEG_ROWS, 16), jnp.float32),
        pltpu.VMEM((5, BURST), jnp.int32),
        pltpu.VMEM((DEG_ROWS_PER_TILE, 16), jnp.float32),
        pltpu.VMEM((BURST, 128), jnp.float32),
        pltpu.VMEM_SHARED((N_PAD, HIDDEN), jnp.float32),
        pltpu.VMEM_SHARED((DEG_ROWS, 16), jnp.float32),
        [pltpu.SemaphoreType.DMA for _ in range(NBUF)],
        [pltpu.SemaphoreType.DMA for _ in range(NBUF)],
    ],
    compiler_params=pltpu.CompilerParams(
        needs_layout_passes=False, use_tc_tiling_on_sc=False),
)
def _sc_agg(m_hbm, ei_hbm, outa_hbm, outd_hbm,
            ev_v, gbufs, ldeg, didx, dbuf, wbuf, agg_sh, deg_sh, gsem, ssem):
    c = lax.axis_index("c")
    s = lax.axis_index("s")
    wid = s * NC + c

    zvec = jnp.zeros((16,), jnp.float32)
    ovec = jnp.ones((16,), jnp.float32)
    i16 = lax.iota(jnp.int32, 16)

    @pl.when(wid < NW - 1)
    def _load_full():
        pltpu.sync_copy(ei_hbm.at[pl.ds(wid * NB, NB)], ev_v)

    @pl.when(wid == NW - 1)
    def _load_tail():
        pltpu.sync_copy(ei_hbm.at[pl.ds((NW - 1) * NB, REAL_TAIL)],
                        ev_v.at[pl.ds(0, REAL_TAIL)])

        def _fill_pad(j, carry):
            base = j * BURST
            for k in range(BURST // 16):
                off = i16 + (base + k * 16)
                ev_v[j, 0, pl.ds(k * 16, 16)] = (
                    N_NODES + lax.rem(off, N_PAD - N_NODES))
                ev_v[j, 1, pl.ds(k * 16, 16)] = lax.rem(off, N_NODES)
            return carry

        lax.fori_loop(REAL_TAIL, NB, _fill_pad, 0)

    def _zero_gbuf0(r, carry):
        for jj in range(HIDDEN // 16):
            gbufs[0][r, pl.ds(jj * 16, 16)] = zvec
        for jj in range(128 // 16):
            wbuf[r, pl.ds(jj * 16, 16)] = zvec
        return carry

    lax.fori_loop(0, BURST, _zero_gbuf0, 0)

    def _zero_ldeg(r, carry):
        ldeg[r, :] = zvec
        return carry

    lax.fori_loop(0, DEG_ROWS, _zero_ldeg, 0)
    for r in range(DEG_ROWS_PER_TILE):
        dbuf[r, :] = zvec
    for r in range(5):
        for k in range(BURST // 16):
            didx[r, pl.ds(k * 16, 16)] = i16 + (r * BURST + k * 16)

    r0 = s * ROWS_PER_TILE
    d0 = s * DEG_ROWS_PER_TILE
    for t in range(4):
        pltpu.sync_copy(gbufs[0], agg_sh.at[pl.ds(r0 + t * BURST, BURST)])
    pltpu.sync_copy(gbufs[0].at[pl.ds(0, ROWS_PER_TILE - 4 * BURST)],
                    agg_sh.at[pl.ds(r0 + 4 * BURST, ROWS_PER_TILE - 4 * BURST)])
    pltpu.sync_copy(dbuf, deg_sh.at[pl.ds(d0, DEG_ROWS_PER_TILE)])
    plsc.subcore_barrier()
    pl.delay(SETTLE_NS)
    plsc.subcore_barrier()

    def _deg_hist(j):
        for k in range(BURST // 16):
            rows16 = ev_v[j, 0, pl.ds(k * 16, 16)]
            rhi = lax.shift_right_logical(rows16, 4)
            rlo = jnp.bitwise_and(rows16, 15)
            plsc.addupdate_scatter(ldeg, [rhi, rlo], ovec)

    for b in range(NBUF):
        pltpu.async_copy(m_hbm.at[ev_v.at[b, 1]], gbufs[b], gsem[b])

    def _octet(i, carry):
        for b in range(NBUF):
            j = i * NBUF + b
            bp = (b - 1) % NBUF
            pltpu.make_async_copy(m_hbm.at[ev_v.at[j, 1]], gbufs[b], gsem[b]).wait()
            pltpu.async_copy(gbufs[b], agg_sh.at[ev_v.at[j, 0]], ssem[b], add=True)
            _deg_hist(j)
            jm1 = jnp.maximum(j - 1, 0)
            if b == 0:
                @pl.when(j >= 1)
                def _wait_prev():
                    pltpu.make_async_copy(
                        gbufs[bp], agg_sh.at[ev_v.at[jm1, 0]], ssem[bp]).wait()
            else:
                pltpu.make_async_copy(
                    gbufs[bp], agg_sh.at[ev_v.at[jm1, 0]], ssem[bp]).wait()

            @pl.when(jnp.logical_and(j >= 1, j + NBUF - 1 < NB))
            def _issue_next():
                pltpu.async_copy(
                    m_hbm.at[ev_v.at[j + NBUF - 1, 1]], gbufs[bp], gsem[bp])
        return carry

    lax.fori_loop(0, NB // NBUF, _octet, 0)
    pltpu.make_async_copy(
        gbufs[(NB - 1) % NBUF], agg_sh.at[ev_v.at[NB - 1, 0]],
        ssem[(NB - 1) % NBUF]).wait()

    for r in range(5):
        pltpu.sync_copy(ldeg.at[pl.ds(r * BURST, BURST)],
                        deg_sh.at[didx.at[r]], add=True)

    plsc.subcore_barrier()
    pltpu.sync_copy(agg_sh.at[pl.ds(r0, BURST)], gbufs[1])
    pl.delay(SETTLE_NS)
    plsc.subcore_barrier()

    for t in range(4):
        pltpu.sync_copy(agg_sh.at[pl.ds(r0 + t * BURST, BURST)],
                        wbuf.at[pl.ds(0, BURST), pl.ds(0, HIDDEN)])
        pltpu.sync_copy(wbuf, outa_hbm.at[c, pl.ds(r0 + t * BURST, BURST)])
    tail = ROWS_PER_TILE - 4 * BURST
    pltpu.sync_copy(agg_sh.at[pl.ds(r0 + 4 * BURST, tail)],
                    wbuf.at[pl.ds(0, tail), pl.ds(0, HIDDEN)])
    pltpu.sync_copy(wbuf.at[pl.ds(0, tail)],
                    outa_hbm.at[c, pl.ds(r0 + 4 * BURST, tail)])
    pltpu.sync_copy(deg_sh.at[pl.ds(d0, DEG_ROWS_PER_TILE)], dbuf)
    pltpu.sync_copy(dbuf, outd_hbm.at[c, pl.ds(d0, DEG_ROWS_PER_TILE)])


def kernel(h, edge_index, W1, b1, W2, b2, W3, b3, W4, b4):
    ei = edge_index.astype(jnp.int32)
    ei3 = ei.reshape(2, EI_ROWS, BURST).transpose(1, 0, 2)

    m_tab = pl.pallas_call(
        _msg_body,
        out_shape=jax.ShapeDtypeStruct((N_NODES, HIDDEN), jnp.float32),
    )(h, W1, b1.reshape(1, HIDDEN), W2, b2.reshape(1, HIDDEN))

    pagg, pdeg = _sc_agg(m_tab, ei3)
    degflat = pdeg.reshape(NC, N_DEG)
    dd0 = degflat[0, :N_NODES, None]
    dd1 = degflat[1, :N_NODES, None]

    out = pl.pallas_call(
        _upd_body,
        out_shape=jax.ShapeDtypeStruct((N_NODES, 2), jnp.float32),
    )(h, pagg, dd0, dd1, W3[:IN_DIM], W3[IN_DIM:], b3.reshape(1, HIDDEN),
      W4, b4.reshape(1, 2))
    return out


# speedup vs baseline: 15.9477x; 1.0604x over previous
import functools

import jax
import jax.numpy as jnp
from jax import lax
from jax.experimental import pallas as pl
from jax.experimental.pallas import tpu as pltpu
from jax.experimental.pallas import tpu_sc as plsc

N_NODES = 10000
N_EDGES = 320000
IN_DIM = 128
HIDDEN = 64

NC = 2
NS = 16
NW = NC * NS

BURST = 128
NBUF = 5
NB = 80
EDGES_PER_TILE = NB * BURST
E_PAD = NW * EDGES_PER_TILE
N_PAD = 10112
ROWS_PER_TILE = N_PAD // NS
DEG_ROWS = 640
N_DEG = DEG_ROWS * 16
DEG_ROWS_PER_TILE = DEG_ROWS // NS
SETTLE_NS = 3000

EI_ROWS = N_EDGES // BURST
REAL_TAIL = EI_ROWS - (NW - 1) * NB


def _msg_body(h_ref, w1_ref, b1_ref, w2_ref, b2_ref, out_ref):
    m1 = jnp.dot(h_ref[...], w1_ref[...], preferred_element_type=jnp.float32)
    m1 = jnp.maximum(m1 + b1_ref[...], 0.0)
    m2 = jnp.dot(m1, w2_ref[...], preferred_element_type=jnp.float32)
    out_ref[...] = jnp.maximum(m2 + b2_ref[...], 0.0)


def _upd_body(h_ref, pa_ref, d0_ref, d1_ref, w3a_ref, w3b_ref, b3_ref, w4_ref,
              b4_ref, out_ref):
    p = (pa_ref[0, :N_NODES, :HIDDEN] + pa_ref[1, :N_NODES, :HIDDEN])
    deg = jnp.maximum(d0_ref[...] + d1_ref[...], 1.0)
    agg = p / deg
    z = jnp.dot(h_ref[...], w3a_ref[...], preferred_element_type=jnp.float32)
    z = z + jnp.dot(agg, w3b_ref[...], preferred_element_type=jnp.float32)
    z = jnp.maximum(z + b3_ref[...], 0.0)
    out_ref[...] = jnp.dot(z, w4_ref[...], preferred_element_type=jnp.float32) + b4_ref[...]


_sc_mesh = plsc.VectorSubcoreMesh(
    core_axis_name="c", subcore_axis_name="s", num_cores=NC, num_subcores=NS)


@functools.partial(
    pl.kernel,
    out_type=(jax.ShapeDtypeStruct((NC, N_PAD, 128), jnp.float32),
              jax.ShapeDtypeStruct((NC, DEG_ROWS, 16), jnp.float32)),
    mesh=_sc_mesh,
    scratch_types=[
        pltpu.VMEM((NB, 2, BURST), jnp.int32),
        [pltpu.VMEM((BURST, HIDDEN), jnp.float32) for _ in range(NBUF)],
        pltpu.VMEM((DEG_ROWS, 16), jnp.float32),
        pltpu.VMEM((5, BURST), jnp.int32),
        pltpu.VMEM((DEG_ROWS_PER_TILE, 16), jnp.float32),
        pltpu.VMEM((BURST, 128), jnp.float32),
        pltpu.VMEM_SHARED((N_PAD, HIDDEN), jnp.float32),
        pltpu.VMEM_SHARED((DEG_ROWS, 16), jnp.float32),
        [pltpu.SemaphoreType.DMA for _ in range(NBUF)],
        [pltpu.SemaphoreType.DMA for _ in range(NBUF)],
    ],
    compiler_params=pltpu.CompilerParams(
        needs_layout_passes=False, use_tc_tiling_on_sc=False),
)
def _sc_agg(m_hbm, ei_hbm, outa_hbm, outd_hbm,
            ev_v, gbufs, ldeg, didx, dbuf, wbuf, agg_sh, deg_sh, gsem, ssem):
    c = lax.axis_index("c")
    s = lax.axis_index("s")
    wid = s * NC + c

    zvec = jnp.zeros((16,), jnp.float32)
    ovec = jnp.ones((16,), jnp.float32)
    i16 = lax.iota(jnp.int32, 16)

    for r in range(DEG_ROWS_PER_TILE):
        dbuf[r, :] = zvec

    def _zero_gbuf0(r, carry):
        for jj in range(HIDDEN // 16):
            gbufs[0][r, pl.ds(jj * 16, 16)] = zvec
        for jj in range(128 // 16):
            wbuf[r, pl.ds(jj * 16, 16)] = zvec
        return carry

    lax.fori_loop(0, BURST, _zero_gbuf0, 0)

    r0 = s * ROWS_PER_TILE
    d0 = s * DEG_ROWS_PER_TILE
    for t in range(4):
        pltpu.sync_copy(gbufs[0], agg_sh.at[pl.ds(r0 + t * BURST, BURST)])
    pltpu.sync_copy(gbufs[0].at[pl.ds(0, ROWS_PER_TILE - 4 * BURST)],
                    agg_sh.at[pl.ds(r0 + 4 * BURST, ROWS_PER_TILE - 4 * BURST)])
    pltpu.sync_copy(dbuf, deg_sh.at[pl.ds(d0, DEG_ROWS_PER_TILE)])

    @pl.when(wid < NW - 1)
    def _load_full():
        pltpu.sync_copy(ei_hbm.at[pl.ds(wid * NB, NB)], ev_v)

    @pl.when(wid == NW - 1)
    def _load_tail():
        pltpu.sync_copy(ei_hbm.at[pl.ds((NW - 1) * NB, REAL_TAIL)],
                        ev_v.at[pl.ds(0, REAL_TAIL)])

        def _fill_pad(j, carry):
            base = j * BURST
            for k in range(BURST // 16):
                off = i16 + (base + k * 16)
                ev_v[j, 0, pl.ds(k * 16, 16)] = (
                    N_NODES + lax.rem(off, N_PAD - N_NODES))
                ev_v[j, 1, pl.ds(k * 16, 16)] = lax.rem(off, N_NODES)
            return carry

        lax.fori_loop(REAL_TAIL, NB, _fill_pad, 0)

    def _zero_ldeg(r, carry):
        ldeg[r, :] = zvec
        return carry

    lax.fori_loop(0, DEG_ROWS, _zero_ldeg, 0)
    for r in range(5):
        for k in range(BURST // 16):
            didx[r, pl.ds(k * 16, 16)] = i16 + (r * BURST + k * 16)

    plsc.subcore_barrier()

    def _deg_hist(j):
        for k in range(BURST // 16):
            rows16 = ev_v[j, 0, pl.ds(k * 16, 16)]
            rhi = lax.shift_right_logical(rows16, 4)
            rlo = jnp.bitwise_and(rows16, 15)
            plsc.addupdate_scatter(ldeg, [rhi, rlo], ovec)

    for b in range(NBUF):
        pltpu.async_copy(m_hbm.at[ev_v.at[b, 1]], gbufs[b], gsem[b])

    def _octet(i, carry):
        for b in range(NBUF):
            j = i * NBUF + b
            bp = (b - 1) % NBUF
            pltpu.make_async_copy(m_hbm.at[ev_v.at[j, 1]], gbufs[b], gsem[b]).wait()
            pltpu.async_copy(gbufs[b], agg_sh.at[ev_v.at[j, 0]], ssem[b], add=True)
            _deg_hist(j)
            jm1 = jnp.maximum(j - 1, 0)
            if b == 0:
                @pl.when(j >= 1)
                def _wait_prev():
                    pltpu.make_async_copy(
                        gbufs[bp], agg_sh.at[ev_v.at[jm1, 0]], ssem[bp]).wait()
            else:
                pltpu.make_async_copy(
                    gbufs[bp], agg_sh.at[ev_v.at[jm1, 0]], ssem[bp]).wait()

            @pl.when(jnp.logical_and(j >= 1, j + NBUF - 1 < NB))
            def _issue_next():
                pltpu.async_copy(
                    m_hbm.at[ev_v.at[j + NBUF - 1, 1]], gbufs[bp], gsem[bp])
        return carry

    lax.fori_loop(0, NB // NBUF, _octet, 0)
    pltpu.make_async_copy(
        gbufs[(NB - 1) % NBUF], agg_sh.at[ev_v.at[NB - 1, 0]],
        ssem[(NB - 1) % NBUF]).wait()

    for r in range(5):
        pltpu.sync_copy(ldeg.at[pl.ds(r * BURST, BURST)],
                        deg_sh.at[didx.at[r]], add=True)

    plsc.subcore_barrier()
    pltpu.sync_copy(agg_sh.at[pl.ds(r0, BURST)], gbufs[1])
    pl.delay(SETTLE_NS)
    plsc.subcore_barrier()

    for t in range(4):
        pltpu.sync_copy(agg_sh.at[pl.ds(r0 + t * BURST, BURST)],
                        wbuf.at[pl.ds(0, BURST), pl.ds(0, HIDDEN)])
        pltpu.sync_copy(wbuf, outa_hbm.at[c, pl.ds(r0 + t * BURST, BURST)])
    tail = ROWS_PER_TILE - 4 * BURST
    pltpu.sync_copy(agg_sh.at[pl.ds(r0 + 4 * BURST, tail)],
                    wbuf.at[pl.ds(0, tail), pl.ds(0, HIDDEN)])
    pltpu.sync_copy(wbuf.at[pl.ds(0, tail)],
                    outa_hbm.at[c, pl.ds(r0 + 4 * BURST, tail)])
    pltpu.sync_copy(deg_sh.at[pl.ds(d0, DEG_ROWS_PER_TILE)], dbuf)
    pltpu.sync_copy(dbuf, outd_hbm.at[c, pl.ds(d0, DEG_ROWS_PER_TILE)])


def kernel(h, edge_index, W1, b1, W2, b2, W3, b3, W4, b4):
    ei = edge_index.astype(jnp.int32)
    ei3 = ei.reshape(2, EI_ROWS, BURST).transpose(1, 0, 2)

    m_tab = pl.pallas_call(
        _msg_body,
        out_shape=jax.ShapeDtypeStruct((N_NODES, HIDDEN), jnp.float32),
    )(h, W1, b1.reshape(1, HIDDEN), W2, b2.reshape(1, HIDDEN))

    pagg, pdeg = _sc_agg(m_tab, ei3)
    degflat = pdeg.reshape(NC, N_DEG)
    dd0 = degflat[0, :N_NODES, None]
    dd1 = degflat[1, :N_NODES, None]

    out = pl.pallas_call(
        _upd_body,
        out_shape=jax.ShapeDtypeStruct((N_NODES, 2), jnp.float32),
    )(h, pagg, dd0, dd1, W3[:IN_DIM], W3[IN_DIM:], b3.reshape(1, HIDDEN),
      W4, b4.reshape(1, 2))
    return out
